# baseline (speedup 1.0000x reference)
import functools

import jax
import jax.numpy as jnp
from jax import lax
from jax.experimental import pallas as pl
from jax.experimental.pallas import tpu as pltpu
from jax.experimental.pallas import tpu_sc as plsc

N_NODES = 10000
D = 128
NC = 2
NS = 16
CHUNK = 128
SPF = 1
EF = SPF * CHUNK
F0 = 104
F1 = 54
ACC_ROWS = 10112
ROWS_OUT = ACC_ROWS // NS

_BCAST_DNUMS = lax.GatherDimensionNumbers(
    offset_dims=(), collapsed_slice_dims=(0,), start_index_map=(0,))


def _sc_agg_body(feat_hbm, src_hbm, dst_hbm, val_hbm, out_hbm,
                 rows_a, rows_b, sidx_a, sidx_b, didx_a, didx_b,
                 val_a, val_b, acc_sh,
                 sg_a, sg_b, ss_a, ss_b, ssi_a, ssi_b,
                 sdi_a, sdi_b, sv_a, sv_b):
    cid = lax.axis_index("c")
    sid = lax.axis_index("s")
    nframes = jnp.where(cid == 0, F0, F1)
    npairs = nframes // 2
    fb = jnp.where(cid == 0, sid * F0, NS * F0 + sid * F1)

    rows = (rows_a, rows_b)
    sidx = (sidx_a, sidx_b)
    didx = (didx_a, didx_b)
    val = (val_a, val_b)
    sg = (sg_a, sg_b)
    ss = (ss_a, ss_b)
    ssi = (ssi_a, ssi_b)
    sdi = (sdi_a, sdi_b)
    sv = (sv_a, sv_b)

    def issue_gather(p):
        for r in range(SPF):
            pltpu.async_copy(feat_hbm.at[sidx[p].at[r]],
                             rows[p].at[pl.ds(r * CHUNK, CHUNK)], sg[p])

    def wait_gather(p):
        for r in range(SPF):
            pltpu.make_async_copy(feat_hbm.at[sidx[p].at[r]],
                                  rows[p].at[pl.ds(r * CHUNK, CHUNK)],
                                  sg[p]).wait()

    def issue_scatter(p):
        for r in range(SPF):
            pltpu.async_copy(rows[p].at[pl.ds(r * CHUNK, CHUNK)],
                             acc_sh.at[didx[p].at[r]], ss[p], add=True)

    def wait_scatter(p):
        for r in range(SPF):
            pltpu.make_async_copy(rows[p].at[pl.ds(r * CHUNK, CHUNK)],
                                  acc_sh.at[didx[p].at[r]], ss[p]).wait()

    def scale(p):
        rv, vv = rows[p], val[p]

        def grp(g, c):
            v16 = vv[pl.ds(g * 16, 16)]
            for l in range(16):
                bidx = jnp.full((16, 1), l, jnp.int32)
                vvl = lax.gather(v16, bidx, _BCAST_DNUMS, (1,),
                                 mode=lax.GatherScatterMode.PROMISE_IN_BOUNDS)
                e = g * 16 + l
                for c8 in range(8):
                    sl = pl.ds(c8 * 16, 16)
                    rv[e, sl] = rv[e, sl] * vvl
            return c
        lax.fori_loop(0, EF // 16, grp, 0)

    def zrow(i, carry):
        for c8 in range(8):
            rows_a[i, pl.ds(c8 * 16, 16)] = jnp.zeros((16,), jnp.float32)
        return carry
    lax.fori_loop(0, EF, zrow, 0)
    zbase = sid * ROWS_OUT
    zcopies, zoff = [], 0
    while zoff < ROWS_OUT:
        zcopies.append((zoff, min(EF, ROWS_OUT - zoff)))
        zoff += zcopies[-1][1]
    for zo, zn in zcopies:
        pltpu.async_copy(rows_a.at[pl.ds(0, zn)],
                         acc_sh.at[pl.ds(zbase + zo, zn)], sg_a)
    for zo, zn in zcopies:
        pltpu.make_async_copy(rows_a.at[pl.ds(0, zn)],
                              acc_sh.at[pl.ds(zbase + zo, zn)], sg_a).wait()
    plsc.subcore_barrier()

    pltpu.async_copy(src_hbm.at[fb], sidx_a, ssi_a)
    pltpu.async_copy(src_hbm.at[fb + 1], sidx_b, ssi_b)
    pltpu.async_copy(val_hbm.at[pl.ds(fb * EF, EF)], val_a, sv_a)
    pltpu.async_copy(val_hbm.at[pl.ds((fb + 1) * EF, EF)], val_b, sv_b)
    pltpu.async_copy(dst_hbm.at[fb], didx_a, sdi_a)
    pltpu.make_async_copy(src_hbm.at[fb], sidx_a, ssi_a).wait()
    issue_gather(0)

    def frame(j, p):
        q = 1 - p
        wait_gather(p)
        @pl.when(j + 2 < nframes)
        def _():
            pltpu.async_copy(src_hbm.at[fb + j + 2], sidx[p], ssi[p])
        @pl.when(j >= 1)
        def _():
            wait_scatter(q)
        @pl.when(j + 1 < nframes)
        def _():
            pltpu.async_copy(dst_hbm.at[fb + j + 1], didx[q], sdi[q])
            pltpu.make_async_copy(src_hbm.at[fb], sidx[q],
                                  ssi[q]).wait()
            issue_gather(q)
        pltpu.make_async_copy(val_hbm.at[pl.ds(fb * EF, EF)], val[p], sv[p]).wait()
        scale(p)
        pltpu.make_async_copy(dst_hbm.at[fb], didx[p], sdi[p]).wait()
        issue_scatter(p)
        @pl.when(j + 2 < nframes)
        def _():
            pltpu.async_copy(val_hbm.at[pl.ds((fb + j + 2) * EF, EF)], val[p], sv[p])

    def pair(t, carry):
        frame(2 * t, 0)
        frame(2 * t + 1, 1)
        return carry
    lax.fori_loop(0, npairs, pair, 0)
    wait_scatter(1)
    plsc.subcore_barrier()

    pltpu.sync_copy(acc_sh.at[pl.ds(sid * ROWS_OUT, ROWS_OUT)],
                    out_hbm.at[cid, pl.ds(sid * ROWS_OUT, ROWS_OUT)])


_sc_agg = functools.partial(
    pl.kernel,
    mesh=plsc.VectorSubcoreMesh(core_axis_name="c", subcore_axis_name="s"),
    out_type=jax.ShapeDtypeStruct((NC, ACC_ROWS, D), jnp.float32),
    scratch_types=[
        pltpu.VMEM((EF, D), jnp.float32),
        pltpu.VMEM((EF, D), jnp.float32),
        pltpu.VMEM((SPF, CHUNK), jnp.int32),
        pltpu.VMEM((SPF, CHUNK), jnp.int32),
        pltpu.VMEM((SPF, CHUNK), jnp.int32),
        pltpu.VMEM((SPF, CHUNK), jnp.int32),
        pltpu.VMEM((EF,), jnp.float32),
        pltpu.VMEM((EF,), jnp.float32),
        pltpu.VMEM_SHARED((ACC_ROWS, D), jnp.float32),
    ] + [pltpu.SemaphoreType.DMA] * 10,
)(_sc_agg_body)


def _combine_body(p_ref, w_ref, b_ref, o_ref):
    a = p_ref[0] + p_ref[1]
    o_ref[...] = jnp.dot(a, w_ref[...],
                         preferred_element_type=jnp.float32) + b_ref[...]


def _combine(partials, W, b):
    blk = 1000
    return pl.pallas_call(
        _combine_body,
        grid=(N_NODES // blk,),
        in_specs=[
            pl.BlockSpec((NC, blk, D), lambda i: (0, i, 0)),
            pl.BlockSpec((D, D), lambda i: (0, 0)),
            pl.BlockSpec((1, D), lambda i: (0, 0)),
        ],
        out_specs=pl.BlockSpec((blk, D), lambda i: (i, 0)),
        out_shape=jax.ShapeDtypeStruct((N_NODES, D), jnp.float32),
    )(partials, W, b.reshape(1, D))


def kernel(adj_edge_index, adj_edge_values, features, W, b):
    src = adj_edge_index[1].astype(jnp.int32)
    dst = adj_edge_index[0].astype(jnp.int32)
    val = adj_edge_values.astype(jnp.float32)
    cap = NS * (F0 + F1) * EF
    pad = cap - src.shape[0]
    src_p = jnp.concatenate([src, jnp.zeros((pad,), jnp.int32)]
                            ).reshape(-1, SPF, CHUNK)
    dst_p = jnp.concatenate([dst, jnp.zeros((pad,), jnp.int32)]
                            ).reshape(-1, SPF, CHUNK)
    val_p = jnp.concatenate([val, jnp.zeros((pad,), jnp.float32)])
    partials = _sc_agg(features, src_p, dst_p, val_p)
    return _combine(partials, W, b)


# speedup vs baseline: 1.4572x; 1.4572x over previous
import functools

import jax
import jax.numpy as jnp
from jax import lax
from jax.experimental import pallas as pl
from jax.experimental.pallas import tpu as pltpu
from jax.experimental.pallas import tpu_sc as plsc

N_NODES = 10000
D = 128
NC = 2
NS = 16
CHUNK = 88
SPF = 2
EF = SPF * CHUNK
F0 = 76
F1 = 38
ACC_ROWS = 10112
ROWS_OUT = ACC_ROWS // NS

_BCAST_DNUMS = lax.GatherDimensionNumbers(
    offset_dims=(), collapsed_slice_dims=(0,), start_index_map=(0,))


def _sc_agg_body(feat_hbm, src_hbm, dst_hbm, val_hbm, out_hbm,
                 rows_a, rows_b, sidx_a, sidx_b, didx_a, didx_b,
                 val_a, val_b, acc_sh,
                 sg_a, sg_b, ss_a, ss_b, ssi_a, ssi_b,
                 sdi_a, sdi_b, sv_a, sv_b):
    cid = lax.axis_index("c")
    sid = lax.axis_index("s")
    nframes = jnp.where(cid == 0, F0, F1)
    npairs = nframes // 2
    fb = jnp.where(cid == 0, sid * F0, NS * F0 + sid * F1)

    rows = (rows_a, rows_b)
    sidx = (sidx_a, sidx_b)
    didx = (didx_a, didx_b)
    val = (val_a, val_b)
    sg = (sg_a, sg_b)
    ss = (ss_a, ss_b)
    ssi = (ssi_a, ssi_b)
    sdi = (sdi_a, sdi_b)
    sv = (sv_a, sv_b)

    def issue_gather(p):
        for r in range(SPF):
            pltpu.async_copy(feat_hbm.at[sidx[p].at[r]],
                             rows[p].at[pl.ds(r * CHUNK, CHUNK)], sg[p])

    def wait_gather(p):
        for r in range(SPF):
            pltpu.make_async_copy(feat_hbm.at[sidx[p].at[r]],
                                  rows[p].at[pl.ds(r * CHUNK, CHUNK)],
                                  sg[p]).wait()

    def issue_scatter(p):
        for r in range(SPF):
            pltpu.async_copy(rows[p].at[pl.ds(r * CHUNK, CHUNK)],
                             acc_sh.at[didx[p].at[r]], ss[p], add=True)

    def wait_scatter(p):
        for r in range(SPF):
            pltpu.make_async_copy(rows[p].at[pl.ds(r * CHUNK, CHUNK)],
                                  acc_sh.at[didx[p].at[r]], ss[p]).wait()

    def scale(p):
        rv, vv = rows[p], val[p]

        def grp(g, c):
            v16 = vv[pl.ds(g * 16, 16)]
            for l in range(16):
                bidx = jnp.full((16, 1), l, jnp.int32)
                vvl = lax.gather(v16, bidx, _BCAST_DNUMS, (1,),
                                 mode=lax.GatherScatterMode.PROMISE_IN_BOUNDS)
                e = g * 16 + l
                for c8 in range(8):
                    sl = pl.ds(c8 * 16, 16)
                    rv[e, sl] = rv[e, sl] * vvl
            return c
        lax.fori_loop(0, EF // 16, grp, 0)

    def zrow(i, carry):
        for c8 in range(8):
            rows_a[i, pl.ds(c8 * 16, 16)] = jnp.zeros((16,), jnp.float32)
        return carry
    lax.fori_loop(0, EF, zrow, 0)
    zbase = sid * ROWS_OUT
    zcopies, zoff = [], 0
    while zoff < ROWS_OUT:
        zcopies.append((zoff, min(EF, ROWS_OUT - zoff)))
        zoff += zcopies[-1][1]
    for zo, zn in zcopies:
        pltpu.async_copy(rows_a.at[pl.ds(0, zn)],
                         acc_sh.at[pl.ds(zbase + zo, zn)], sg_a)
    for zo, zn in zcopies:
        pltpu.make_async_copy(rows_a.at[pl.ds(0, zn)],
                              acc_sh.at[pl.ds(zbase + zo, zn)], sg_a).wait()
    plsc.subcore_barrier()

    pltpu.async_copy(src_hbm.at[fb], sidx_a, ssi_a)
    pltpu.async_copy(src_hbm.at[fb + 1], sidx_b, ssi_b)
    pltpu.async_copy(val_hbm.at[pl.ds(fb * EF, EF)], val_a, sv_a)
    pltpu.async_copy(val_hbm.at[pl.ds((fb + 1) * EF, EF)], val_b, sv_b)
    pltpu.async_copy(dst_hbm.at[fb], didx_a, sdi_a)
    pltpu.make_async_copy(src_hbm.at[fb], sidx_a, ssi_a).wait()
    issue_gather(0)

    def frame(j, p):
        q = 1 - p
        wait_gather(p)
        @pl.when(j + 2 < nframes)
        def _():
            pltpu.async_copy(src_hbm.at[fb + j + 2], sidx[p], ssi[p])
        @pl.when(j >= 1)
        def _():
            wait_scatter(q)
        @pl.when(j + 1 < nframes)
        def _():
            pltpu.async_copy(dst_hbm.at[fb + j + 1], didx[q], sdi[q])
            pltpu.make_async_copy(src_hbm.at[fb], sidx[q],
                                  ssi[q]).wait()
            issue_gather(q)
        pltpu.make_async_copy(val_hbm.at[pl.ds(fb * EF, EF)], val[p], sv[p]).wait()
        scale(p)
        pltpu.make_async_copy(dst_hbm.at[fb], didx[p], sdi[p]).wait()
        issue_scatter(p)
        @pl.when(j + 2 < nframes)
        def _():
            pltpu.async_copy(val_hbm.at[pl.ds((fb + j + 2) * EF, EF)], val[p], sv[p])

    def pair(t, carry):
        frame(2 * t, 0)
        frame(2 * t + 1, 1)
        return carry
    lax.fori_loop(0, npairs, pair, 0)
    wait_scatter(1)
    plsc.subcore_barrier()

    pltpu.sync_copy(acc_sh.at[pl.ds(sid * ROWS_OUT, ROWS_OUT)],
                    out_hbm.at[cid, pl.ds(sid * ROWS_OUT, ROWS_OUT)])


_sc_agg = functools.partial(
    pl.kernel,
    mesh=plsc.VectorSubcoreMesh(core_axis_name="c", subcore_axis_name="s"),
    out_type=jax.ShapeDtypeStruct((NC, ACC_ROWS, D), jnp.float32),
    scratch_types=[
        pltpu.VMEM((EF, D), jnp.float32),
        pltpu.VMEM((EF, D), jnp.float32),
        pltpu.VMEM((SPF, CHUNK), jnp.int32),
        pltpu.VMEM((SPF, CHUNK), jnp.int32),
        pltpu.VMEM((SPF, CHUNK), jnp.int32),
        pltpu.VMEM((SPF, CHUNK), jnp.int32),
        pltpu.VMEM((EF,), jnp.float32),
        pltpu.VMEM((EF,), jnp.float32),
        pltpu.VMEM_SHARED((ACC_ROWS, D), jnp.float32),
    ] + [pltpu.SemaphoreType.DMA] * 10,
)(_sc_agg_body)


def _combine_body(p_ref, w_ref, b_ref, o_ref):
    a = p_ref[0] + p_ref[1]
    o_ref[...] = jnp.dot(a, w_ref[...],
                         preferred_element_type=jnp.float32) + b_ref[...]


def _combine(partials, W, b):
    blk = 1000
    return pl.pallas_call(
        _combine_body,
        grid=(N_NODES // blk,),
        in_specs=[
            pl.BlockSpec((NC, blk, D), lambda i: (0, i, 0)),
            pl.BlockSpec((D, D), lambda i: (0, 0)),
            pl.BlockSpec((1, D), lambda i: (0, 0)),
        ],
        out_specs=pl.BlockSpec((blk, D), lambda i: (i, 0)),
        out_shape=jax.ShapeDtypeStruct((N_NODES, D), jnp.float32),
    )(partials, W, b.reshape(1, D))


def kernel(adj_edge_index, adj_edge_values, features, W, b):
    src = adj_edge_index[1].astype(jnp.int32)
    dst = adj_edge_index[0].astype(jnp.int32)
    val = adj_edge_values.astype(jnp.float32)
    cap = NS * (F0 + F1) * EF
    pad = cap - src.shape[0]
    src_p = jnp.concatenate([src, jnp.zeros((pad,), jnp.int32)]
                            ).reshape(-1, SPF, CHUNK)
    dst_p = jnp.concatenate([dst, jnp.zeros((pad,), jnp.int32)]
                            ).reshape(-1, SPF, CHUNK)
    val_p = jnp.concatenate([val, jnp.zeros((pad,), jnp.float32)])
    partials = _sc_agg(features, src_p, dst_p, val_p)
    return _combine(partials, W, b)


# speedup vs baseline: 1.5126x; 1.0380x over previous
import functools

import jax
import jax.numpy as jnp
from jax import lax
from jax.experimental import pallas as pl
from jax.experimental.pallas import tpu as pltpu
from jax.experimental.pallas import tpu_sc as plsc

N_NODES = 10000
D = 128
NC = 2
NS = 16
CHUNK = 88
SPF = 2
EF = SPF * CHUNK
F0 = 72
F1 = 42
ACC_ROWS = 10112
ROWS_OUT = ACC_ROWS // NS

_BCAST_DNUMS = lax.GatherDimensionNumbers(
    offset_dims=(), collapsed_slice_dims=(0,), start_index_map=(0,))


def _sc_agg_body(feat_hbm, src_hbm, dst_hbm, val_hbm, out_hbm,
                 rows_a, rows_b, sidx_a, sidx_b, didx_a, didx_b,
                 val_a, val_b, acc_sh,
                 sg_a, sg_b, ss_a, ss_b, ssi_a, ssi_b,
                 sdi_a, sdi_b, sv_a, sv_b):
    cid = lax.axis_index("c")
    sid = lax.axis_index("s")
    nframes = jnp.where(cid == 0, F0, F1)
    npairs = nframes // 2
    fb = jnp.where(cid == 0, sid * F0, NS * F0 + sid * F1)

    rows = (rows_a, rows_b)
    sidx = (sidx_a, sidx_b)
    didx = (didx_a, didx_b)
    val = (val_a, val_b)
    sg = (sg_a, sg_b)
    ss = (ss_a, ss_b)
    ssi = (ssi_a, ssi_b)
    sdi = (sdi_a, sdi_b)
    sv = (sv_a, sv_b)

    def issue_gather(p):
        for r in range(SPF):
            pltpu.async_copy(feat_hbm.at[sidx[p].at[r]],
                             rows[p].at[pl.ds(r * CHUNK, CHUNK)], sg[p])

    def wait_gather(p):
        for r in range(SPF):
            pltpu.make_async_copy(feat_hbm.at[sidx[p].at[r]],
                                  rows[p].at[pl.ds(r * CHUNK, CHUNK)],
                                  sg[p]).wait()

    def issue_scatter(p):
        for r in range(SPF):
            pltpu.async_copy(rows[p].at[pl.ds(r * CHUNK, CHUNK)],
                             acc_sh.at[didx[p].at[r]], ss[p], add=True)

    def wait_scatter(p):
        for r in range(SPF):
            pltpu.make_async_copy(rows[p].at[pl.ds(r * CHUNK, CHUNK)],
                                  acc_sh.at[didx[p].at[r]], ss[p]).wait()

    def scale(p):
        rv, vv = rows[p], val[p]

        def grp(g, c):
            v16 = vv[pl.ds(g * 16, 16)]
            for l in range(16):
                bidx = jnp.full((16, 1), l, jnp.int32)
                vvl = lax.gather(v16, bidx, _BCAST_DNUMS, (1,),
                                 mode=lax.GatherScatterMode.PROMISE_IN_BOUNDS)
                e = g * 16 + l
                for c8 in range(8):
                    sl = pl.ds(c8 * 16, 16)
                    rv[e, sl] = rv[e, sl] * vvl
            return c
        lax.fori_loop(0, EF // 16, grp, 0)

    def zrow(i, carry):
        for c8 in range(8):
            rows_a[i, pl.ds(c8 * 16, 16)] = jnp.zeros((16,), jnp.float32)
        return carry
    lax.fori_loop(0, EF, zrow, 0)
    zbase = sid * ROWS_OUT
    zcopies, zoff = [], 0
    while zoff < ROWS_OUT:
        zcopies.append((zoff, min(EF, ROWS_OUT - zoff)))
        zoff += zcopies[-1][1]
    for zo, zn in zcopies:
        pltpu.async_copy(rows_a.at[pl.ds(0, zn)],
                         acc_sh.at[pl.ds(zbase + zo, zn)], sg_a)
    for zo, zn in zcopies:
        pltpu.make_async_copy(rows_a.at[pl.ds(0, zn)],
                              acc_sh.at[pl.ds(zbase + zo, zn)], sg_a).wait()
    plsc.subcore_barrier()

    pltpu.async_copy(src_hbm.at[fb], sidx_a, ssi_a)
    pltpu.async_copy(src_hbm.at[fb + 1], sidx_b, ssi_b)
    pltpu.async_copy(val_hbm.at[pl.ds(fb * EF, EF)], val_a, sv_a)
    pltpu.async_copy(val_hbm.at[pl.ds((fb + 1) * EF, EF)], val_b, sv_b)
    pltpu.async_copy(dst_hbm.at[fb], didx_a, sdi_a)
    pltpu.make_async_copy(src_hbm.at[fb], sidx_a, ssi_a).wait()
    issue_gather(0)

    def frame(j, p):
        q = 1 - p
        wait_gather(p)
        @pl.when(j + 2 < nframes)
        def _():
            pltpu.async_copy(src_hbm.at[fb + j + 2], sidx[p], ssi[p])
        @pl.when(j >= 1)
        def _():
            wait_scatter(q)
        @pl.when(j + 1 < nframes)
        def _():
            pltpu.async_copy(dst_hbm.at[fb + j + 1], didx[q], sdi[q])
            pltpu.make_async_copy(src_hbm.at[fb], sidx[q],
                                  ssi[q]).wait()
            issue_gather(q)
        pltpu.make_async_copy(val_hbm.at[pl.ds(fb * EF, EF)], val[p], sv[p]).wait()
        scale(p)
        pltpu.make_async_copy(dst_hbm.at[fb], didx[p], sdi[p]).wait()
        issue_scatter(p)
        @pl.when(j + 2 < nframes)
        def _():
            pltpu.async_copy(val_hbm.at[pl.ds((fb + j + 2) * EF, EF)], val[p], sv[p])

    def pair(t, carry):
        frame(2 * t, 0)
        frame(2 * t + 1, 1)
        return carry
    lax.fori_loop(0, npairs, pair, 0)
    wait_scatter(1)
    plsc.subcore_barrier()

    pltpu.sync_copy(acc_sh.at[pl.ds(sid * ROWS_OUT, ROWS_OUT)],
                    out_hbm.at[cid, pl.ds(sid * ROWS_OUT, ROWS_OUT)])


_sc_agg = functools.partial(
    pl.kernel,
    mesh=plsc.VectorSubcoreMesh(core_axis_name="c", subcore_axis_name="s"),
    out_type=jax.ShapeDtypeStruct((NC, ACC_ROWS, D), jnp.float32),
    scratch_types=[
        pltpu.VMEM((EF, D), jnp.float32),
        pltpu.VMEM((EF, D), jnp.float32),
        pltpu.VMEM((SPF, CHUNK), jnp.int32),
        pltpu.VMEM((SPF, CHUNK), jnp.int32),
        pltpu.VMEM((SPF, CHUNK), jnp.int32),
        pltpu.VMEM((SPF, CHUNK), jnp.int32),
        pltpu.VMEM((EF,), jnp.float32),
        pltpu.VMEM((EF,), jnp.float32),
        pltpu.VMEM_SHARED((ACC_ROWS, D), jnp.float32),
    ] + [pltpu.SemaphoreType.DMA] * 10,
)(_sc_agg_body)


def _combine_body(p_ref, w_ref, b_ref, o_ref):
    a = p_ref[0] + p_ref[1]
    o_ref[...] = jnp.dot(a, w_ref[...],
                         preferred_element_type=jnp.float32) + b_ref[...]


def _combine(partials, W, b):
    blk = 1000
    return pl.pallas_call(
        _combine_body,
        grid=(N_NODES // blk,),
        in_specs=[
            pl.BlockSpec((NC, blk, D), lambda i: (0, i, 0)),
            pl.BlockSpec((D, D), lambda i: (0, 0)),
            pl.BlockSpec((1, D), lambda i: (0, 0)),
        ],
        out_specs=pl.BlockSpec((blk, D), lambda i: (i, 0)),
        out_shape=jax.ShapeDtypeStruct((N_NODES, D), jnp.float32),
    )(partials, W, b.reshape(1, D))


def kernel(adj_edge_index, adj_edge_values, features, W, b):
    src = adj_edge_index[1].astype(jnp.int32)
    dst = adj_edge_index[0].astype(jnp.int32)
    val = adj_edge_values.astype(jnp.float32)
    cap = NS * (F0 + F1) * EF
    pad = cap - src.shape[0]
    src_p = jnp.concatenate([src, jnp.zeros((pad,), jnp.int32)]
                            ).reshape(-1, SPF, CHUNK)
    dst_p = jnp.concatenate([dst, jnp.zeros((pad,), jnp.int32)]
                            ).reshape(-1, SPF, CHUNK)
    val_p = jnp.concatenate([val, jnp.zeros((pad,), jnp.float32)])
    partials = _sc_agg(features, src_p, dst_p, val_p)
    return _combine(partials, W, b)
